# baseline (speedup 1.0000x reference)
import jax
import jax.numpy as jnp
from jax import lax
from jax.experimental import pallas as pl
from jax.experimental.pallas import tpu as pltpu
from jax.experimental.pallas import tpu_sc as plsc

N_NODES = 10000
N_EDGES = 320000
IN_DIM = 200
HIDDEN = 64
OUT_DIM = 16
NUM_GRAPHS = 64

NC = 2
NS = 16
LANES = 16
N_PAD = 10240
ROWS_PER_TILE = N_PAD // NS

HALF = N_PAD // 2
ACC_ROWS = 5248
ACC_PER_TILE = ACC_ROWS // NS

CHUNK = 128
E_PER_TILE = 20480
E_PAD = NS * E_PER_TILE
NCHUNK = E_PER_TILE // CHUNK
NBUF = 4

HROWS = N_PAD // 128
HR_PER_TILE = HROWS // NS


def _dinv_body(dst_hbm, out_hbm, didx_v, hist_v, iota_v, sem, acc_sh):
    cid = lax.axis_index("c")
    sid = lax.axis_index("s")

    @pl.when(cid == 0)
    def _():
        edges_per_tile = N_EDGES // NS

        def zfill(r, carry):
            for c in range(128 // LANES):
                hist_v[r, pl.ds(c * LANES, LANES)] = jnp.zeros((LANES,), jnp.float32)
            return carry

        lax.fori_loop(0, HROWS, zfill, 0)
        for i in range(HROWS // LANES):
            iota_v[pl.ds(i * LANES, LANES)] = (
                lax.iota(jnp.int32, LANES) + i * LANES)
        pltpu.sync_copy(hist_v.at[pl.ds(sid * HR_PER_TILE, HR_PER_TILE)],
                        acc_sh.at[pl.ds(sid * HR_PER_TILE, HR_PER_TILE)])
        pltpu.sync_copy(dst_hbm.at[pl.ds(sid * edges_per_tile, edges_per_tile)],
                        didx_v)

        ones = jnp.full((LANES,), 1.0, jnp.float32)

        def count(i, carry):
            idx = didx_v[pl.ds(i * LANES, LANES)]
            plsc.addupdate_scatter(hist_v, [idx >> 7, idx & 127], ones)
            return carry

        lax.fori_loop(0, edges_per_tile // LANES, count, 0)
        plsc.subcore_barrier()
        pltpu.sync_copy(hist_v, acc_sh.at[iota_v], add=True)
        plsc.subcore_barrier()

        pltpu.sync_copy(acc_sh.at[pl.ds(sid * HR_PER_TILE, HR_PER_TILE)],
                        hist_v.at[pl.ds(0, HR_PER_TILE)])

        def newton(r, carry):
            for c in range(128 // LANES):
                d = hist_v[r, pl.ds(c * LANES, LANES)] + 1.0
                bits = lax.bitcast_convert_type(d, jnp.int32)
                y = lax.bitcast_convert_type(
                    jnp.int32(0x5F3759DF) - (bits >> 1), jnp.float32)
                for _ in range(3):
                    y = y * (1.5 - 0.5 * d * y * y)
                hist_v[r, pl.ds(c * LANES, LANES)] = y
            return carry

        lax.fori_loop(0, HR_PER_TILE, newton, 0)
        pltpu.sync_copy(hist_v.at[pl.ds(0, HR_PER_TILE)],
                        out_hbm.at[pl.ds(sid * HR_PER_TILE, HR_PER_TILE)])


def _agg_body(u_hbm, src_hbm, dst_hbm, out_hbm, sidx_v, didx_v,
              rows0, rows1, rows2, rows3, stage_v,
              semg0, semg1, semg2, semg3, acc_sh):
    cid = lax.axis_index("c")
    sid = lax.axis_index("s")
    rows = (rows0, rows1, rows2, rows3)
    semg = (semg0, semg1, semg2, semg3)

    def zfill(r, carry):
        for c in range(HIDDEN // LANES):
            stage_v[r, pl.ds(c * LANES, LANES)] = jnp.zeros((LANES,), jnp.float32)
        return carry

    lax.fori_loop(0, ACC_PER_TILE, zfill, 0)
    pltpu.sync_copy(stage_v, acc_sh.at[pl.ds(sid * ACC_PER_TILE, ACC_PER_TILE), :])
    pltpu.sync_copy(src_hbm.at[sid], sidx_v)
    pltpu.sync_copy(dst_hbm.at[sid], didx_v)

    base = cid * HALF

    def remap(r, carry):
        for c in range(CHUNK // LANES):
            d = didx_v[r, pl.ds(c * LANES, LANES)] - base
            ok = (d >= 0) & (d < HALF)
            didx_v[r, pl.ds(c * LANES, LANES)] = jnp.where(ok, d, HALF)
        return carry

    lax.fori_loop(0, NCHUNK, remap, 0)
    plsc.subcore_barrier()

    for b in range(NBUF):
        pltpu.async_copy(u_hbm.at[sidx_v.at[b]], rows[b], semg[b])

    def rounds(t, carry):
        for b in range(NBUF):
            j = NBUF * t + b
            pltpu.make_async_copy(u_hbm.at[sidx_v.at[j]], rows[b], semg[b]).wait()
            pltpu.sync_copy(rows[b], acc_sh.at[didx_v.at[j]], add=True)

            @pl.when(t < NCHUNK // NBUF - 1)
            def _():
                pltpu.async_copy(u_hbm.at[sidx_v.at[j + NBUF]], rows[b], semg[b])
        return carry

    lax.fori_loop(0, NCHUNK // NBUF, rounds, 0)
    plsc.subcore_barrier()

    pltpu.sync_copy(acc_sh.at[pl.ds(sid * ACC_PER_TILE, ACC_PER_TILE), :],
                    out_hbm.at[cid, pl.ds(sid * ACC_PER_TILE, ACC_PER_TILE), :])


_SC_PARAMS = pltpu.CompilerParams(use_tc_tiling_on_sc=False,
                                  needs_layout_passes=False)


def _sc_mesh():
    return plsc.VectorSubcoreMesh(core_axis_name="c", subcore_axis_name="s",
                                  num_cores=NC, num_subcores=NS)


def _dinv_call(dst):
    k = pl.kernel(
        _dinv_body,
        out_type=jax.ShapeDtypeStruct((HROWS, 128), jnp.float32),
        mesh=_sc_mesh(),
        compiler_params=_SC_PARAMS,
        scratch_types=[
            pltpu.VMEM((N_EDGES // NS,), jnp.int32),
            pltpu.VMEM((HROWS, 128), jnp.float32),
            pltpu.VMEM((HROWS,), jnp.int32),
            pltpu.SemaphoreType.DMA,
            pltpu.VMEM_SHARED((HROWS, 128), jnp.float32),
        ],
    )
    return k(dst)


def _agg_call(u, src3, dst3):
    k = pl.kernel(
        _agg_body,
        out_type=jax.ShapeDtypeStruct((NC, ACC_ROWS, HIDDEN), jnp.float32),
        mesh=_sc_mesh(),
        compiler_params=_SC_PARAMS,
        scratch_types=[
            pltpu.VMEM((NCHUNK, CHUNK), jnp.int32),
            pltpu.VMEM((NCHUNK, CHUNK), jnp.int32),
            pltpu.VMEM((CHUNK, HIDDEN), jnp.float32),
            pltpu.VMEM((CHUNK, HIDDEN), jnp.float32),
            pltpu.VMEM((CHUNK, HIDDEN), jnp.float32),
            pltpu.VMEM((CHUNK, HIDDEN), jnp.float32),
            pltpu.VMEM((ACC_PER_TILE, HIDDEN), jnp.float32),
            pltpu.SemaphoreType.DMA,
            pltpu.SemaphoreType.DMA,
            pltpu.SemaphoreType.DMA,
            pltpu.SemaphoreType.DMA,
            pltpu.VMEM_SHARED((ACC_ROWS, HIDDEN), jnp.float32),
        ],
    )
    return k(u, src3, dst3)


def _agg_of(p_ref):
    return jnp.concatenate(
        [p_ref[0][:HALF], p_ref[1][:N_NODES - HALF]], axis=0)


def _tc1_body(x_ref, w_ref, dinv_ref, u_ref):
    xw = jnp.dot(x_ref[...], w_ref[...], preferred_element_type=jnp.float32)
    u_ref[:N_NODES] = xw * dinv_ref[...]


def _tc2_body(p_ref, u_ref, dinv_ref, b_ref, w_ref, u2_ref):
    agg = _agg_of(p_ref) + u_ref[:N_NODES]
    h = jnp.maximum(dinv_ref[...] * agg + b_ref[...], 0.0)
    u2_ref[:N_NODES] = jnp.dot(h, w_ref[...],
                               preferred_element_type=jnp.float32) * dinv_ref[...]


def _tc3_body(p_ref, u_ref, dinv_ref, b_ref, batch_ref, wc_ref, bc_ref, out_ref):
    agg = _agg_of(p_ref) + u_ref[:N_NODES]
    h = jnp.maximum(dinv_ref[...] * agg + b_ref[...], 0.0)
    gid = lax.broadcasted_iota(jnp.int32, (NUM_GRAPHS, N_NODES), 0)
    onehot_t = (batch_ref[...] == gid).astype(jnp.float32)
    seg = jnp.dot(onehot_t, h, preferred_element_type=jnp.float32)
    counts = jnp.sum(onehot_t, axis=1, keepdims=True)
    hg = seg / jnp.maximum(counts, 1.0)
    out_ref[...] = jnp.dot(hg, wc_ref[...],
                           preferred_element_type=jnp.float32) + bc_ref[...]


def kernel(x, edge_index, batch, W1, b1, W2, b2, Wc, bc):
    npad = E_PAD - N_EDGES
    srcp = jnp.concatenate(
        [edge_index[0], jnp.zeros((npad,), jnp.int32)]).reshape(NS, NCHUNK, CHUNK)
    dstp = jnp.concatenate(
        [edge_index[1], jnp.full((npad,), N_PAD, jnp.int32)]).reshape(NS, NCHUNK, CHUNK)

    dinv_pad = _dinv_call(edge_index[1])
    dinv = dinv_pad.reshape(N_PAD)[:N_NODES, None]

    u1 = pl.pallas_call(
        _tc1_body,
        out_shape=jax.ShapeDtypeStruct((N_PAD, HIDDEN), jnp.float32),
    )(x, W1, dinv)

    p1 = _agg_call(u1, srcp, dstp)

    u2 = pl.pallas_call(
        _tc2_body,
        out_shape=jax.ShapeDtypeStruct((N_PAD, HIDDEN), jnp.float32),
    )(p1, u1, dinv, b1.reshape(1, HIDDEN), W2)

    p2 = _agg_call(u2, srcp, dstp)

    out = pl.pallas_call(
        _tc3_body,
        out_shape=jax.ShapeDtypeStruct((NUM_GRAPHS, OUT_DIM), jnp.float32),
    )(p2, u2, dinv, b2.reshape(1, HIDDEN), batch.reshape(1, N_NODES),
      Wc, bc.reshape(1, OUT_DIM))
    return out


# speedup vs baseline: 1.6857x; 1.6857x over previous
import jax
import jax.numpy as jnp
from jax import lax
from jax.experimental import pallas as pl
from jax.experimental.pallas import tpu as pltpu
from jax.experimental.pallas import tpu_sc as plsc

N_NODES = 10000
N_EDGES = 320000
IN_DIM = 200
HIDDEN = 64
OUT_DIM = 16
NUM_GRAPHS = 64

NC = 2
NS = 16
LANES = 16
N_PAD = 10240

ACC_A = 632
ACC_B = N_NODES - 15 * ACC_A

CHUNK = 128
E_PER_TILE = 10240
E_PAD = NC * NS * E_PER_TILE
NCHUNK = E_PER_TILE // CHUNK
NBUF = 2

HROWS = N_PAD // 128


DEG_PER_TILE = E_PAD // NS
DEG_NCHUNK = DEG_PER_TILE // CHUNK
DEG_ROWS = N_PAD // NS


def _deg_body(dst_hbm, out_hbm, didx_v, ones_v, buf_v, acc_sh):
    cid = lax.axis_index("c")
    sid = lax.axis_index("s")

    @pl.when(cid == 0)
    def _():
        for i in range(CHUNK // LANES):
            ones_v[pl.ds(i * LANES, LANES)] = jnp.full((LANES,), 1.0, jnp.float32)

        def zfill(i, carry):
            buf_v[pl.ds(i * LANES, LANES)] = jnp.zeros((LANES,), jnp.float32)
            return carry

        lax.fori_loop(0, DEG_ROWS // LANES, zfill, 0)
        pltpu.sync_copy(buf_v, acc_sh.at[pl.ds(sid * DEG_ROWS, DEG_ROWS)])
        pltpu.sync_copy(dst_hbm.at[sid], didx_v)
        plsc.subcore_barrier()

        def count(j, carry):
            pltpu.sync_copy(ones_v, acc_sh.at[didx_v.at[j]], add=True)
            return carry

        lax.fori_loop(0, DEG_NCHUNK, count, 0)
        plsc.subcore_barrier()

        pltpu.sync_copy(acc_sh.at[pl.ds(sid * DEG_ROWS, DEG_ROWS)], buf_v)

        def newton(i, carry):
            d = buf_v[pl.ds(i * LANES, LANES)] + 1.0
            bits = lax.bitcast_convert_type(d, jnp.int32)
            y = lax.bitcast_convert_type(
                jnp.int32(0x5F3759DF) - (bits >> 1), jnp.float32)
            for _ in range(3):
                y = y * (1.5 - 0.5 * d * y * y)
            buf_v[pl.ds(i * LANES, LANES)] = y
            return carry

        lax.fori_loop(0, DEG_ROWS // LANES, newton, 0)
        pltpu.sync_copy(buf_v, out_hbm.at[pl.ds(sid * DEG_ROWS, DEG_ROWS)])


def _agg_body(u_hbm, src_hbm, dst_hbm, out_hbm, sidx_v, didx_v,
              rows0, rows1, stage_v,
              semg0, semg1, acc_sh):
    cid = lax.axis_index("c")
    sid = lax.axis_index("s")
    wid = cid * NS + sid
    rows = (rows0, rows1)
    semg = (semg0, semg1)

    def zfill(r, carry):
        for c in range(HIDDEN // LANES):
            stage_v[r, pl.ds(c * LANES, LANES)] = jnp.zeros((LANES,), jnp.float32)
        return carry

    lax.fori_loop(0, ACC_A, zfill, 0)

    @pl.when(sid < 15)
    def _():
        pltpu.sync_copy(stage_v, acc_sh.at[pl.ds(sid * ACC_A, ACC_A), :])

    @pl.when(sid == 15)
    def _():
        pltpu.sync_copy(stage_v.at[pl.ds(0, ACC_B), :],
                        acc_sh.at[pl.ds(15 * ACC_A, ACC_B), :])

    pltpu.sync_copy(src_hbm.at[wid], sidx_v)
    pltpu.sync_copy(dst_hbm.at[wid], didx_v)
    plsc.subcore_barrier()

    for b in range(NBUF):
        pltpu.async_copy(u_hbm.at[sidx_v.at[b]], rows[b], semg[b])

    def rounds(t, carry):
        for b in range(NBUF):
            j = NBUF * t + b
            pltpu.make_async_copy(u_hbm.at[sidx_v.at[j]], rows[b], semg[b]).wait()
            pltpu.sync_copy(rows[b], acc_sh.at[didx_v.at[j]], add=True)

            @pl.when(t < NCHUNK // NBUF - 1)
            def _():
                pltpu.async_copy(u_hbm.at[sidx_v.at[j + NBUF]], rows[b], semg[b])
        return carry

    lax.fori_loop(0, NCHUNK // NBUF, rounds, 0)
    plsc.subcore_barrier()

    @pl.when(sid < 15)
    def _():
        pltpu.sync_copy(acc_sh.at[pl.ds(sid * ACC_A, ACC_A), :],
                        out_hbm.at[cid, pl.ds(sid * ACC_A, ACC_A), :])

    @pl.when(sid == 15)
    def _():
        pltpu.sync_copy(acc_sh.at[pl.ds(15 * ACC_A, ACC_B), :],
                        out_hbm.at[cid, pl.ds(15 * ACC_A, ACC_B), :])


_SC_PARAMS = pltpu.CompilerParams(use_tc_tiling_on_sc=False)
_SC_PARAMS_NLP = pltpu.CompilerParams(use_tc_tiling_on_sc=False,
                                      needs_layout_passes=False)


def _sc_mesh():
    return plsc.VectorSubcoreMesh(core_axis_name="c", subcore_axis_name="s",
                                  num_cores=NC, num_subcores=NS)


def _deg_call(dst3):
    k = pl.kernel(
        _deg_body,
        out_type=jax.ShapeDtypeStruct((N_PAD,), jnp.float32),
        mesh=_sc_mesh(),
        compiler_params=_SC_PARAMS,
        scratch_types=[
            pltpu.VMEM((DEG_NCHUNK, CHUNK), jnp.int32),
            pltpu.VMEM((CHUNK,), jnp.float32),
            pltpu.VMEM((DEG_ROWS,), jnp.float32),
            pltpu.VMEM_SHARED((N_PAD,), jnp.float32),
        ],
    )
    return k(dst3)


def _agg_call(u, src3, dst3):
    k = pl.kernel(
        _agg_body,
        out_type=jax.ShapeDtypeStruct((NC, N_NODES, HIDDEN), jnp.float32),
        mesh=_sc_mesh(),
        compiler_params=_SC_PARAMS,
        scratch_types=[
            pltpu.VMEM((NCHUNK, CHUNK), jnp.int32),
            pltpu.VMEM((NCHUNK, CHUNK), jnp.int32),
            pltpu.VMEM((CHUNK, HIDDEN), jnp.float32),
            pltpu.VMEM((CHUNK, HIDDEN), jnp.float32),
            pltpu.VMEM((ACC_A, HIDDEN), jnp.float32),
            pltpu.SemaphoreType.DMA,
            pltpu.SemaphoreType.DMA,
            pltpu.VMEM_SHARED((N_NODES, HIDDEN), jnp.float32),
        ],
    )
    return k(u, src3, dst3)


def _tc1_body(x_ref, w_ref, dinv_ref, u_ref):
    xw = jnp.dot(x_ref[...], w_ref[...], preferred_element_type=jnp.float32)
    u_ref[:N_NODES] = xw * dinv_ref[...]
    u_ref[N_NODES:] = jnp.zeros((N_PAD - N_NODES, HIDDEN), jnp.float32)


def _tc2_body(p_ref, u_ref, dinv_ref, b_ref, w_ref, u2_ref):
    agg = p_ref[0] + p_ref[1] + u_ref[:N_NODES]
    h = jnp.maximum(dinv_ref[...] * agg + b_ref[...], 0.0)
    u2_ref[:N_NODES] = jnp.dot(h, w_ref[...],
                               preferred_element_type=jnp.float32) * dinv_ref[...]
    u2_ref[N_NODES:] = jnp.zeros((N_PAD - N_NODES, HIDDEN), jnp.float32)


def _tc3_body(p_ref, u_ref, dinv_ref, b_ref, batch_ref, wc_ref, bc_ref, out_ref):
    agg = p_ref[0] + p_ref[1] + u_ref[:N_NODES]
    h = jnp.maximum(dinv_ref[...] * agg + b_ref[...], 0.0)
    gid = lax.broadcasted_iota(jnp.int32, (NUM_GRAPHS, N_NODES), 0)
    onehot_t = (batch_ref[...] == gid).astype(jnp.float32)
    seg = jnp.dot(onehot_t, h, preferred_element_type=jnp.float32)
    counts = jnp.sum(onehot_t, axis=1, keepdims=True)
    hg = seg / jnp.maximum(counts, 1.0)
    out_ref[...] = jnp.dot(hg, wc_ref[...],
                           preferred_element_type=jnp.float32) + bc_ref[...]


def kernel(x, edge_index, batch, W1, b1, W2, b2, Wc, bc):
    npad = E_PAD - N_EDGES
    srcp = jnp.concatenate(
        [edge_index[0], jnp.full((npad,), N_NODES, jnp.int32)]
    ).reshape(NC * NS, NCHUNK, CHUNK)
    dstp = jnp.concatenate(
        [edge_index[1], jnp.zeros((npad,), jnp.int32)]
    ).reshape(NC * NS, NCHUNK, CHUNK)
    dstdeg = jnp.concatenate(
        [edge_index[1], jnp.full((npad,), N_NODES + 200, jnp.int32)]
    ).reshape(NS, DEG_NCHUNK, CHUNK)

    dinv_pad = _deg_call(dstdeg)
    dinv = dinv_pad[:N_NODES, None]

    u1 = pl.pallas_call(
        _tc1_body,
        out_shape=jax.ShapeDtypeStruct((N_PAD, HIDDEN), jnp.float32),
    )(x, W1, dinv)

    p1 = _agg_call(u1, srcp, dstp)

    u2 = pl.pallas_call(
        _tc2_body,
        out_shape=jax.ShapeDtypeStruct((N_PAD, HIDDEN), jnp.float32),
    )(p1, u1, dinv, b1.reshape(1, HIDDEN), W2)

    p2 = _agg_call(u2, srcp, dstp)

    out = pl.pallas_call(
        _tc3_body,
        out_shape=jax.ShapeDtypeStruct((NUM_GRAPHS, OUT_DIM), jnp.float32),
    )(p2, u2, dinv, b2.reshape(1, HIDDEN), batch.reshape(1, N_NODES),
      Wc, bc.reshape(1, OUT_DIM))
    return out


# speedup vs baseline: 3.5359x; 2.0976x over previous
import jax
import jax.numpy as jnp
from jax import lax
from jax.experimental import pallas as pl
from jax.experimental.pallas import tpu as pltpu
from jax.experimental.pallas import tpu_sc as plsc

N_NODES = 10000
N_EDGES = 320000
IN_DIM = 200
HIDDEN = 64
OUT_DIM = 16
NUM_GRAPHS = 64

NC = 2
NS = 16
LANES = 16
N_PAD = 10240
ROWS_PER_TILE = N_PAD // NS

ACC_A = 632
ACC_B = N_NODES - 15 * ACC_A

CHUNK = 128
E_PER_TILE = 10240
E_PAD = NC * NS * E_PER_TILE
NCHUNK = E_PER_TILE // CHUNK
NBUF = 2

HROWS = N_PAD // 128


DEG_PER_TILE = E_PAD // NS
DEG_NCHUNK = DEG_PER_TILE // CHUNK
DEG_ROWS = N_PAD // NS


def _deg_body(dst_hbm, out_hbm, didx_v, ones_v, buf_v, acc_sh):
    cid = lax.axis_index("c")
    sid = lax.axis_index("s")

    @pl.when(cid == 0)
    def _():
        for i in range(CHUNK // LANES):
            ones_v[pl.ds(i * LANES, LANES)] = jnp.full((LANES,), 1.0, jnp.float32)

        def zfill(i, carry):
            buf_v[pl.ds(i * LANES, LANES)] = jnp.zeros((LANES,), jnp.float32)
            return carry

        lax.fori_loop(0, DEG_ROWS // LANES, zfill, 0)
        pltpu.sync_copy(buf_v, acc_sh.at[pl.ds(sid * DEG_ROWS, DEG_ROWS)])
        pltpu.sync_copy(dst_hbm.at[sid], didx_v)
        plsc.subcore_barrier()

        def count(j, carry):
            pltpu.sync_copy(ones_v, acc_sh.at[didx_v.at[j]], add=True)
            return carry

        lax.fori_loop(0, DEG_NCHUNK, count, 0)
        plsc.subcore_barrier()

        pltpu.sync_copy(acc_sh.at[pl.ds(sid * DEG_ROWS, DEG_ROWS)], buf_v)

        def newton(i, carry):
            d = buf_v[pl.ds(i * LANES, LANES)] + 1.0
            bits = lax.bitcast_convert_type(d, jnp.int32)
            y = lax.bitcast_convert_type(
                jnp.int32(0x5F3759DF) - (bits >> 1), jnp.float32)
            for _ in range(3):
                y = y * (1.5 - 0.5 * d * y * y)
            buf_v[pl.ds(i * LANES, LANES)] = y
            return carry

        lax.fori_loop(0, DEG_ROWS // LANES, newton, 0)
        pltpu.sync_copy(buf_v, out_hbm.at[pl.ds(sid * DEG_ROWS, DEG_ROWS)])


def _agg_body(u_hbm, src_hbm, dst_hbm, out_hbm, sidx_v, didx_v,
              rows0, rows1, zbuf_v,
              semg0, semg1, u_sh, acc_sh):
    cid = lax.axis_index("c")
    sid = lax.axis_index("s")
    wid = cid * NS + sid
    rows = (rows0, rows1)
    semg = (semg0, semg1)

    for r in range(8):
        for c in range(HIDDEN // LANES):
            zbuf_v[r, pl.ds(c * LANES, LANES)] = jnp.zeros((LANES,), jnp.float32)

    def zcopy(i, carry):
        pltpu.sync_copy(zbuf_v, acc_sh.at[pl.ds(sid * ACC_A + 8 * i, 8), :])
        return carry

    @pl.when(sid < 15)
    def _():
        lax.fori_loop(0, ACC_A // 8, zcopy, 0)

    @pl.when(sid == 15)
    def _():
        lax.fori_loop(0, ACC_B // 8, zcopy, 0)

    pltpu.sync_copy(u_hbm.at[pl.ds(sid * ROWS_PER_TILE, ROWS_PER_TILE), :],
                    u_sh.at[pl.ds(sid * ROWS_PER_TILE, ROWS_PER_TILE), :])
    pltpu.sync_copy(src_hbm.at[wid], sidx_v)
    pltpu.sync_copy(dst_hbm.at[wid], didx_v)
    plsc.subcore_barrier()

    for b in range(NBUF):
        pltpu.async_copy(u_sh.at[sidx_v.at[b]], rows[b], semg[b])

    def rounds(t, carry):
        for b in range(NBUF):
            j = NBUF * t + b
            pltpu.make_async_copy(u_sh.at[sidx_v.at[j]], rows[b], semg[b]).wait()
            pltpu.sync_copy(rows[b], acc_sh.at[didx_v.at[j]], add=True)

            @pl.when(t < NCHUNK // NBUF - 1)
            def _():
                pltpu.async_copy(u_sh.at[sidx_v.at[j + NBUF]], rows[b], semg[b])
        return carry

    lax.fori_loop(0, NCHUNK // NBUF, rounds, 0)
    plsc.subcore_barrier()

    @pl.when(sid < 15)
    def _():
        pltpu.sync_copy(acc_sh.at[pl.ds(sid * ACC_A, ACC_A), :],
                        out_hbm.at[cid, pl.ds(sid * ACC_A, ACC_A), :])

    @pl.when(sid == 15)
    def _():
        pltpu.sync_copy(acc_sh.at[pl.ds(15 * ACC_A, ACC_B), :],
                        out_hbm.at[cid, pl.ds(15 * ACC_A, ACC_B), :])


_SC_PARAMS = pltpu.CompilerParams(use_tc_tiling_on_sc=False)
_SC_PARAMS_NLP = pltpu.CompilerParams(use_tc_tiling_on_sc=False,
                                      needs_layout_passes=False)


def _sc_mesh():
    return plsc.VectorSubcoreMesh(core_axis_name="c", subcore_axis_name="s",
                                  num_cores=NC, num_subcores=NS)


def _deg_call(dst3):
    k = pl.kernel(
        _deg_body,
        out_type=jax.ShapeDtypeStruct((N_PAD,), jnp.float32),
        mesh=_sc_mesh(),
        compiler_params=_SC_PARAMS,
        scratch_types=[
            pltpu.VMEM((DEG_NCHUNK, CHUNK), jnp.int32),
            pltpu.VMEM((CHUNK,), jnp.float32),
            pltpu.VMEM((DEG_ROWS,), jnp.float32),
            pltpu.VMEM_SHARED((N_PAD,), jnp.float32),
        ],
    )
    return k(dst3)


def _agg_call(u, src3, dst3):
    k = pl.kernel(
        _agg_body,
        out_type=jax.ShapeDtypeStruct((NC, N_NODES, HIDDEN), jnp.float32),
        mesh=_sc_mesh(),
        compiler_params=_SC_PARAMS,
        scratch_types=[
            pltpu.VMEM((NCHUNK, CHUNK), jnp.int32),
            pltpu.VMEM((NCHUNK, CHUNK), jnp.int32),
            pltpu.VMEM((CHUNK, HIDDEN), jnp.float32),
            pltpu.VMEM((CHUNK, HIDDEN), jnp.float32),
            pltpu.VMEM((8, HIDDEN), jnp.float32),
            pltpu.SemaphoreType.DMA,
            pltpu.SemaphoreType.DMA,
            pltpu.VMEM_SHARED((N_PAD, HIDDEN), jnp.float32),
            pltpu.VMEM_SHARED((N_NODES, HIDDEN), jnp.float32),
        ],
    )
    return k(u, src3, dst3)


def _tc1_body(x_ref, w_ref, dinv_ref, u_ref):
    xw = jnp.dot(x_ref[...], w_ref[...], preferred_element_type=jnp.float32)
    u_ref[:N_NODES] = xw * dinv_ref[...]
    u_ref[N_NODES:] = jnp.zeros((N_PAD - N_NODES, HIDDEN), jnp.float32)


def _tc2_body(p_ref, u_ref, dinv_ref, b_ref, w_ref, u2_ref):
    agg = p_ref[0] + p_ref[1] + u_ref[:N_NODES]
    h = jnp.maximum(dinv_ref[...] * agg + b_ref[...], 0.0)
    u2_ref[:N_NODES] = jnp.dot(h, w_ref[...],
                               preferred_element_type=jnp.float32) * dinv_ref[...]
    u2_ref[N_NODES:] = jnp.zeros((N_PAD - N_NODES, HIDDEN), jnp.float32)


def _tc3_body(p_ref, u_ref, dinv_ref, b_ref, batch_ref, wc_ref, bc_ref, out_ref):
    agg = p_ref[0] + p_ref[1] + u_ref[:N_NODES]
    h = jnp.maximum(dinv_ref[...] * agg + b_ref[...], 0.0)
    gid = lax.broadcasted_iota(jnp.int32, (NUM_GRAPHS, N_NODES), 0)
    onehot_t = (batch_ref[...] == gid).astype(jnp.float32)
    seg = jnp.dot(onehot_t, h, preferred_element_type=jnp.float32)
    counts = jnp.sum(onehot_t, axis=1, keepdims=True)
    hg = seg / jnp.maximum(counts, 1.0)
    out_ref[...] = jnp.dot(hg, wc_ref[...],
                           preferred_element_type=jnp.float32) + bc_ref[...]


def kernel(x, edge_index, batch, W1, b1, W2, b2, Wc, bc):
    npad = E_PAD - N_EDGES
    srcp = jnp.concatenate(
        [edge_index[0], jnp.full((npad,), N_NODES, jnp.int32)]
    ).reshape(NC * NS, NCHUNK, CHUNK)
    dstp = jnp.concatenate(
        [edge_index[1], jnp.zeros((npad,), jnp.int32)]
    ).reshape(NC * NS, NCHUNK, CHUNK)
    dstdeg = jnp.concatenate(
        [edge_index[1], jnp.full((npad,), N_NODES + 200, jnp.int32)]
    ).reshape(NS, DEG_NCHUNK, CHUNK)

    dinv_pad = _deg_call(dstdeg)
    dinv = dinv_pad[:N_NODES, None]

    u1 = pl.pallas_call(
        _tc1_body,
        out_shape=jax.ShapeDtypeStruct((N_PAD, HIDDEN), jnp.float32),
    )(x, W1, dinv)

    p1 = _agg_call(u1, srcp, dstp)

    u2 = pl.pallas_call(
        _tc2_body,
        out_shape=jax.ShapeDtypeStruct((N_PAD, HIDDEN), jnp.float32),
    )(p1, u1, dinv, b1.reshape(1, HIDDEN), W2)

    p2 = _agg_call(u2, srcp, dstp)

    out = pl.pallas_call(
        _tc3_body,
        out_shape=jax.ShapeDtypeStruct((NUM_GRAPHS, OUT_DIM), jnp.float32),
    )(p2, u2, dinv, b2.reshape(1, HIDDEN), batch.reshape(1, N_NODES),
      Wc, bc.reshape(1, OUT_DIM))
    return out


# speedup vs baseline: 3.5890x; 1.0150x over previous
import jax
import jax.numpy as jnp
from jax import lax
from jax.experimental import pallas as pl
from jax.experimental.pallas import tpu as pltpu
from jax.experimental.pallas import tpu_sc as plsc

N_NODES = 10000
N_EDGES = 320000
IN_DIM = 200
HIDDEN = 64
OUT_DIM = 16
NUM_GRAPHS = 64

NC = 2
NS = 16
LANES = 16
N_PAD = 10240
ROWS_PER_TILE = N_PAD // NS

ACC_A = 632
ACC_B = N_NODES - 15 * ACC_A

CHUNK = 112
E_PER_TILE = 10304
E_PAD = NC * NS * E_PER_TILE
NCHUNK = E_PER_TILE // CHUNK
NBUF = 4
U_ROWS = 10112

HROWS = N_PAD // 128


DEG_PER_TILE = E_PAD // NS
DEG_NCHUNK = DEG_PER_TILE // CHUNK
DEG_ROWS = N_PAD // NS


def _deg_body(dst_hbm, out_hbm, didx_v, ones_v, buf_v, acc_sh):
    cid = lax.axis_index("c")
    sid = lax.axis_index("s")

    @pl.when(cid == 0)
    def _():
        for i in range(CHUNK // LANES):
            ones_v[pl.ds(i * LANES, LANES)] = jnp.full((LANES,), 1.0, jnp.float32)

        def zfill(i, carry):
            buf_v[pl.ds(i * LANES, LANES)] = jnp.zeros((LANES,), jnp.float32)
            return carry

        lax.fori_loop(0, DEG_ROWS // LANES, zfill, 0)
        pltpu.sync_copy(buf_v, acc_sh.at[pl.ds(sid * DEG_ROWS, DEG_ROWS)])
        pltpu.sync_copy(dst_hbm.at[sid], didx_v)
        plsc.subcore_barrier()

        def count(j, carry):
            pltpu.sync_copy(ones_v, acc_sh.at[didx_v.at[j]], add=True)
            return carry

        lax.fori_loop(0, DEG_NCHUNK, count, 0)
        plsc.subcore_barrier()

        pltpu.sync_copy(acc_sh.at[pl.ds(sid * DEG_ROWS, DEG_ROWS)], buf_v)

        def newton(i, carry):
            d = buf_v[pl.ds(i * LANES, LANES)] + 1.0
            bits = lax.bitcast_convert_type(d, jnp.int32)
            y = lax.bitcast_convert_type(
                jnp.int32(0x5F3759DF) - (bits >> 1), jnp.float32)
            for _ in range(3):
                y = y * (1.5 - 0.5 * d * y * y)
            buf_v[pl.ds(i * LANES, LANES)] = y
            return carry

        lax.fori_loop(0, DEG_ROWS // LANES, newton, 0)
        pltpu.sync_copy(buf_v, out_hbm.at[pl.ds(sid * DEG_ROWS, DEG_ROWS)])


def _agg_body(u_hbm, src_hbm, dst_hbm, out_hbm, sidx_v, didx_v,
              rows0, rows1, rows2, rows3, zbuf_v,
              semg0, semg1, semg2, semg3, u_sh, acc_sh):
    cid = lax.axis_index("c")
    sid = lax.axis_index("s")
    wid = cid * NS + sid
    rows = (rows0, rows1, rows2, rows3)
    semg = (semg0, semg1, semg2, semg3)

    for r in range(8):
        for c in range(HIDDEN // LANES):
            zbuf_v[r, pl.ds(c * LANES, LANES)] = jnp.zeros((LANES,), jnp.float32)

    def zcopy(i, carry):
        pltpu.sync_copy(zbuf_v, acc_sh.at[pl.ds(sid * ACC_A + 8 * i, 8), :])
        return carry

    @pl.when(sid < 15)
    def _():
        lax.fori_loop(0, ACC_A // 8, zcopy, 0)

    @pl.when(sid == 15)
    def _():
        lax.fori_loop(0, ACC_B // 8, zcopy, 0)

    pltpu.sync_copy(u_hbm.at[pl.ds(sid * (U_ROWS // NS), U_ROWS // NS), :],
                    u_sh.at[pl.ds(sid * (U_ROWS // NS), U_ROWS // NS), :])
    pltpu.sync_copy(src_hbm.at[wid], sidx_v)
    pltpu.sync_copy(dst_hbm.at[wid], didx_v)
    plsc.subcore_barrier()

    for b in range(NBUF):
        pltpu.async_copy(u_sh.at[sidx_v.at[b]], rows[b], semg[b])

    def rounds(t, carry):
        for b in range(NBUF):
            j = NBUF * t + b
            pltpu.make_async_copy(u_sh.at[sidx_v.at[j]], rows[b], semg[b]).wait()
            pltpu.sync_copy(rows[b], acc_sh.at[didx_v.at[j]], add=True)

            @pl.when(t < NCHUNK // NBUF - 1)
            def _():
                pltpu.async_copy(u_sh.at[sidx_v.at[j + NBUF]], rows[b], semg[b])
        return carry

    lax.fori_loop(0, NCHUNK // NBUF, rounds, 0)
    plsc.subcore_barrier()

    @pl.when(sid < 15)
    def _():
        pltpu.sync_copy(acc_sh.at[pl.ds(sid * ACC_A, ACC_A), :],
                        out_hbm.at[cid, pl.ds(sid * ACC_A, ACC_A), :])

    @pl.when(sid == 15)
    def _():
        pltpu.sync_copy(acc_sh.at[pl.ds(15 * ACC_A, ACC_B), :],
                        out_hbm.at[cid, pl.ds(15 * ACC_A, ACC_B), :])


_SC_PARAMS = pltpu.CompilerParams(use_tc_tiling_on_sc=False)
_SC_PARAMS_NLP = pltpu.CompilerParams(use_tc_tiling_on_sc=False,
                                      needs_layout_passes=False)


def _sc_mesh():
    return plsc.VectorSubcoreMesh(core_axis_name="c", subcore_axis_name="s",
                                  num_cores=NC, num_subcores=NS)


def _deg_call(dst3):
    k = pl.kernel(
        _deg_body,
        out_type=jax.ShapeDtypeStruct((N_PAD,), jnp.float32),
        mesh=_sc_mesh(),
        compiler_params=_SC_PARAMS,
        scratch_types=[
            pltpu.VMEM((DEG_NCHUNK, CHUNK), jnp.int32),
            pltpu.VMEM((CHUNK,), jnp.float32),
            pltpu.VMEM((DEG_ROWS,), jnp.float32),
            pltpu.VMEM_SHARED((N_PAD,), jnp.float32),
        ],
    )
    return k(dst3)


def _agg_call(u, src3, dst3):
    k = pl.kernel(
        _agg_body,
        out_type=jax.ShapeDtypeStruct((NC, N_NODES, HIDDEN), jnp.float32),
        mesh=_sc_mesh(),
        compiler_params=_SC_PARAMS,
        scratch_types=[
            pltpu.VMEM((NCHUNK, CHUNK), jnp.int32),
            pltpu.VMEM((NCHUNK, CHUNK), jnp.int32),
            pltpu.VMEM((CHUNK, HIDDEN), jnp.float32),
            pltpu.VMEM((CHUNK, HIDDEN), jnp.float32),
            pltpu.VMEM((CHUNK, HIDDEN), jnp.float32),
            pltpu.VMEM((CHUNK, HIDDEN), jnp.float32),
            pltpu.VMEM((8, HIDDEN), jnp.float32),
            pltpu.SemaphoreType.DMA,
            pltpu.SemaphoreType.DMA,
            pltpu.SemaphoreType.DMA,
            pltpu.SemaphoreType.DMA,
            pltpu.VMEM_SHARED((U_ROWS, HIDDEN), jnp.float32),
            pltpu.VMEM_SHARED((N_NODES, HIDDEN), jnp.float32),
        ],
    )
    return k(u, src3, dst3)


def _tc1a_body(x_ref, w_ref, xw_ref):
    xw_ref[:N_NODES] = jnp.dot(x_ref[...], w_ref[...],
                               preferred_element_type=jnp.float32)
    xw_ref[N_NODES:] = jnp.zeros((N_PAD - N_NODES, HIDDEN), jnp.float32)


def _tc1b_body(xw_ref, dinv_ref, u_ref):
    u_ref[...] = xw_ref[...] * dinv_ref[...]


def _tc2_body(p_ref, u_ref, dinv_ref, b_ref, w_ref, u2_ref):
    agg = p_ref[0] + p_ref[1] + u_ref[:N_NODES]
    h = jnp.maximum(dinv_ref[...] * agg + b_ref[...], 0.0)
    u2_ref[:N_NODES] = jnp.dot(h, w_ref[...],
                               preferred_element_type=jnp.float32) * dinv_ref[...]
    u2_ref[N_NODES:] = jnp.zeros((N_PAD - N_NODES, HIDDEN), jnp.float32)


def _tc3_body(p_ref, u_ref, dinv_ref, b_ref, batch_ref, wc_ref, bc_ref, out_ref):
    agg = p_ref[0] + p_ref[1] + u_ref[:N_NODES]
    h = jnp.maximum(dinv_ref[...] * agg + b_ref[...], 0.0)
    gid = lax.broadcasted_iota(jnp.int32, (NUM_GRAPHS, N_NODES), 0)
    onehot_t = (batch_ref[...] == gid).astype(jnp.float32)
    seg = jnp.dot(onehot_t, h, preferred_element_type=jnp.float32)
    counts = jnp.sum(onehot_t, axis=1, keepdims=True)
    hg = seg / jnp.maximum(counts, 1.0)
    out_ref[...] = jnp.dot(hg, wc_ref[...],
                           preferred_element_type=jnp.float32) + bc_ref[...]


def kernel(x, edge_index, batch, W1, b1, W2, b2, Wc, bc):
    npad = E_PAD - N_EDGES
    srcp = jnp.concatenate(
        [edge_index[0], jnp.full((npad,), N_NODES, jnp.int32)]
    ).reshape(NC * NS, NCHUNK, CHUNK)
    dstp = jnp.concatenate(
        [edge_index[1], jnp.zeros((npad,), jnp.int32)]
    ).reshape(NC * NS, NCHUNK, CHUNK)
    dstdeg = jnp.concatenate(
        [edge_index[1], jnp.full((npad,), N_NODES + 200, jnp.int32)]
    ).reshape(NS, DEG_NCHUNK, CHUNK)

    dinv_pad = _deg_call(dstdeg)
    dinv = dinv_pad[:N_NODES, None]

    xw1 = pl.pallas_call(
        _tc1a_body,
        out_shape=jax.ShapeDtypeStruct((N_PAD, HIDDEN), jnp.float32),
    )(x, W1)
    u1 = pl.pallas_call(
        _tc1b_body,
        out_shape=jax.ShapeDtypeStruct((N_PAD, HIDDEN), jnp.float32),
    )(xw1, dinv_pad[:, None])

    p1 = _agg_call(u1, srcp, dstp)

    u2 = pl.pallas_call(
        _tc2_body,
        out_shape=jax.ShapeDtypeStruct((N_PAD, HIDDEN), jnp.float32),
    )(p1, u1, dinv, b1.reshape(1, HIDDEN), W2)

    p2 = _agg_call(u2, srcp, dstp)

    out = pl.pallas_call(
        _tc3_body,
        out_shape=jax.ShapeDtypeStruct((NUM_GRAPHS, OUT_DIM), jnp.float32),
    )(p2, u2, dinv, b2.reshape(1, HIDDEN), batch.reshape(1, N_NODES),
      Wc, bc.reshape(1, OUT_DIM))
    return out
